# baseline (speedup 1.0000x reference)
import functools

import jax
import jax.numpy as jnp
from jax import lax
from jax.experimental import pallas as pl
from jax.experimental.pallas import tpu as pltpu
from jax.experimental.pallas import tpu_sc as plsc

FG_THRESH = 0.9
BG_THRESH = 0.1

_ROWS = 32768
_COLS = 512
_NC, _NS, _L = 2, 16, 16
_NW = _NC * _NS
_ROWS_PER_W = _ROWS // _NW
_CH_ROWS = 32
_NCHUNK = _ROWS_PER_W // _CH_ROWS

_mesh = plsc.VectorSubcoreMesh(core_axis_name="c", subcore_axis_name="s")


@functools.partial(
    pl.kernel,
    out_type=jax.ShapeDtypeStruct((_ROWS, _COLS), jnp.float32),
    mesh=_mesh,
    scratch_types=[
        pltpu.VMEM((_CH_ROWS, _COLS), jnp.float32),
        pltpu.VMEM((_CH_ROWS, _COLS), jnp.float32),
        pltpu.VMEM((_CH_ROWS, _COLS), jnp.float32),
        pltpu.VMEM((_CH_ROWS, _COLS), jnp.float32),
        pltpu.SemaphoreType.DMA,
        pltpu.SemaphoreType.DMA,
        pltpu.SemaphoreType.DMA,
        pltpu.SemaphoreType.DMA,
    ],
)
def _sc_fuse(t_hbm, a_hbm, out_hbm, t0, a0, t1, a1, ls0, ls1, ss0, ss1):
    wid = lax.axis_index("s") * _NC + lax.axis_index("c")
    base = wid * _ROWS_PER_W
    bufs = ((t0, a0, ls0, ss0), (t1, a1, ls1, ss1))

    def start_loads(g, tb, ab, sem):
        off = base + g * _CH_ROWS
        pltpu.async_copy(t_hbm.at[pl.ds(off, _CH_ROWS), :], tb, sem)
        pltpu.async_copy(a_hbm.at[pl.ds(off, _CH_ROWS), :], ab, sem)

    def wait_loads(tb, ab, sem):
        pltpu.make_async_copy(t_hbm.at[pl.ds(base, _CH_ROWS), :], tb, sem).wait()
        pltpu.make_async_copy(a_hbm.at[pl.ds(base, _CH_ROWS), :], ab, sem).wait()

    def wait_store(ab, sem):
        pltpu.make_async_copy(ab, out_hbm.at[pl.ds(base, _CH_ROWS), :], sem).wait()

    start_loads(0, t0, a0, ls0)

    def pair_body(p, carry):
        for b in range(2):
            g = 2 * p + b
            tb, ab, ls, ss = bufs[b]
            tn, an, ln, sn = bufs[1 - b]

            @pl.when(g >= 1)
            def _w():
                wait_store(an, sn)

            @pl.when(g + 1 < _NCHUNK)
            def _s():
                start_loads(g + 1, tn, an, ln)

            wait_loads(tb, ab, ls)

            @plsc.parallel_loop(0, _CH_ROWS, 1, unroll=2)
            def row_body(r):
                for c in range(0, _COLS, _L):
                    t = tb[r, pl.ds(c, _L)]
                    a = ab[r, pl.ds(c, _L)]
                    keep = (t > FG_THRESH) | (t < BG_THRESH)
                    ab[r, pl.ds(c, _L)] = jnp.where(keep, t, a)

            out_off = base + g * _CH_ROWS
            pltpu.async_copy(ab, out_hbm.at[pl.ds(out_off, _CH_ROWS), :], ss)
        return carry

    lax.fori_loop(0, _NCHUNK // 2, pair_body, 0)
    wait_store(a1, ss1)


def kernel(trimap, alpha):
    t = trimap.reshape(_ROWS, _COLS)
    a = alpha.reshape(_ROWS, _COLS)
    out = _sc_fuse(t, a)
    return out.reshape(trimap.shape)


# speedup vs baseline: 1.0821x; 1.0821x over previous
import functools

import jax
import jax.numpy as jnp
from jax import lax
from jax.experimental import pallas as pl
from jax.experimental.pallas import tpu as pltpu
from jax.experimental.pallas import tpu_sc as plsc

FG_THRESH = 0.9
BG_THRESH = 0.1

_ROWS = 32768
_COLS = 512
_NC, _NS, _L = 2, 16, 16
_NW = _NC * _NS
_ROWS_PER_W = _ROWS // _NW
_CH_ROWS = 32
_NCHUNK = _ROWS_PER_W // _CH_ROWS

_mesh = plsc.VectorSubcoreMesh(core_axis_name="c", subcore_axis_name="s")


@functools.partial(
    pl.kernel,
    out_type=jax.ShapeDtypeStruct((_ROWS, _COLS), jnp.float32),
    mesh=_mesh,
    scratch_types=[
        pltpu.VMEM((_CH_ROWS, _COLS), jnp.float32),
        pltpu.VMEM((_CH_ROWS, _COLS), jnp.float32),
        pltpu.VMEM((_CH_ROWS, _COLS), jnp.float32),
        pltpu.VMEM((_CH_ROWS, _COLS), jnp.float32),
        pltpu.SemaphoreType.DMA,
        pltpu.SemaphoreType.DMA,
        pltpu.SemaphoreType.DMA,
        pltpu.SemaphoreType.DMA,
    ],
)
def _sc_fuse(t_hbm, a_hbm, out_hbm, t0, a0, t1, a1, ls0, ls1, ss0, ss1):
    wid = lax.axis_index("s") * _NC + lax.axis_index("c")
    base = wid * _ROWS_PER_W
    bufs = ((t0, a0, ls0, ss0), (t1, a1, ls1, ss1))

    def start_loads(g, tb, ab, sem):
        off = base + g * _CH_ROWS
        pltpu.async_copy(t_hbm.at[pl.ds(off, _CH_ROWS), :], tb, sem)
        pltpu.async_copy(a_hbm.at[pl.ds(off, _CH_ROWS), :], ab, sem)

    def wait_loads(tb, ab, sem):
        pltpu.make_async_copy(t_hbm.at[pl.ds(base, _CH_ROWS), :], tb, sem).wait()
        pltpu.make_async_copy(a_hbm.at[pl.ds(base, _CH_ROWS), :], ab, sem).wait()

    def wait_store(ab, sem):
        pltpu.make_async_copy(ab, out_hbm.at[pl.ds(base, _CH_ROWS), :], sem).wait()

    start_loads(0, t0, a0, ls0)

    def pair_body(p, carry):
        for b in range(2):
            g = 2 * p + b
            tb, ab, ls, ss = bufs[b]
            tn, an, ln, sn = bufs[1 - b]

            @pl.when(g >= 1)
            def _w():
                wait_store(an, sn)

            @pl.when(g + 1 < _NCHUNK)
            def _s():
                start_loads(g + 1, tn, an, ln)

            wait_loads(tb, ab, ls)

            def row_body(r, carry2):
                @plsc.parallel_loop(0, _COLS, _L, unroll=16)
                def vec_body(c):
                    t = tb[r, pl.ds(c, _L)]
                    a = ab[r, pl.ds(c, _L)]
                    keep = (t > FG_THRESH) | (t < BG_THRESH)
                    ab[r, pl.ds(c, _L)] = jnp.where(keep, t, a)

                return carry2

            lax.fori_loop(0, _CH_ROWS, row_body, 0)

            out_off = base + g * _CH_ROWS
            pltpu.async_copy(ab, out_hbm.at[pl.ds(out_off, _CH_ROWS), :], ss)
        return carry

    lax.fori_loop(0, _NCHUNK // 2, pair_body, 0)
    wait_store(a1, ss1)


def kernel(trimap, alpha):
    t = trimap.reshape(_ROWS, _COLS)
    a = alpha.reshape(_ROWS, _COLS)
    out = _sc_fuse(t, a)
    return out.reshape(trimap.shape)


# speedup vs baseline: 1.1715x; 1.0826x over previous
import functools

import jax
import jax.numpy as jnp
from jax import lax
from jax.experimental import pallas as pl
from jax.experimental.pallas import tpu as pltpu
from jax.experimental.pallas import tpu_sc as plsc

FG_THRESH = 0.9
BG_THRESH = 0.1

_ROWS = 32768
_COLS = 512
_NC, _NS, _L = 2, 16, 16
_NW = _NC * _NS
_ROWS_PER_W = _ROWS // _NW
_CH_ROWS = 16
_NCHUNK = _ROWS_PER_W // _CH_ROWS
_NSLOT = 4

_mesh = plsc.VectorSubcoreMesh(core_axis_name="c", subcore_axis_name="s")

_vmem = [pltpu.VMEM((_CH_ROWS, _COLS), jnp.float32) for _ in range(2 * _NSLOT)]
_sems = [pltpu.SemaphoreType.DMA for _ in range(2 * _NSLOT)]


@functools.partial(
    pl.kernel,
    out_type=jax.ShapeDtypeStruct((_ROWS, _COLS), jnp.float32),
    mesh=_mesh,
    scratch_types=_vmem + _sems,
)
def _sc_fuse(t_hbm, a_hbm, out_hbm, *scratch):
    tbufs = scratch[0:_NSLOT]
    abufs = scratch[_NSLOT:2 * _NSLOT]
    lsems = scratch[2 * _NSLOT:3 * _NSLOT]
    ssems = scratch[3 * _NSLOT:4 * _NSLOT]
    wid = lax.axis_index("s") * _NC + lax.axis_index("c")
    base = wid * _ROWS_PER_W

    def start_loads(g, s):
        off = base + g * _CH_ROWS
        pltpu.async_copy(t_hbm.at[pl.ds(off, _CH_ROWS), :], tbufs[s], lsems[s])
        pltpu.async_copy(a_hbm.at[pl.ds(off, _CH_ROWS), :], abufs[s], lsems[s])

    def wait_loads(s):
        pltpu.make_async_copy(t_hbm.at[pl.ds(base, _CH_ROWS), :], tbufs[s], lsems[s]).wait()
        pltpu.make_async_copy(a_hbm.at[pl.ds(base, _CH_ROWS), :], abufs[s], lsems[s]).wait()

    def wait_store(s):
        pltpu.make_async_copy(abufs[s], out_hbm.at[pl.ds(base, _CH_ROWS), :], ssems[s]).wait()

    start_loads(0, 0)
    start_loads(1, 1)

    def quad_body(p, carry):
        for b in range(_NSLOT):
            g = _NSLOT * p + b
            s = b
            nxt = (b + 2) % _NSLOT

            @pl.when((g >= 2) & (g + 2 < _NCHUNK))
            def _w():
                wait_store(nxt)

            @pl.when(g + 2 < _NCHUNK)
            def _s():
                start_loads(g + 2, nxt)

            wait_loads(s)

            def row_body(r, carry2):
                @plsc.parallel_loop(0, _COLS, _L, unroll=8)
                def vec_body(c):
                    t = tbufs[s][r, pl.ds(c, _L)]
                    a = abufs[s][r, pl.ds(c, _L)]
                    keep = (t > FG_THRESH) | (t < BG_THRESH)
                    abufs[s][r, pl.ds(c, _L)] = jnp.where(keep, t, a)

                return carry2

            lax.fori_loop(0, _CH_ROWS, row_body, 0)

            out_off = base + g * _CH_ROWS
            pltpu.async_copy(abufs[s], out_hbm.at[pl.ds(out_off, _CH_ROWS), :], ssems[s])
        return carry

    lax.fori_loop(0, _NCHUNK // _NSLOT, quad_body, 0)
    for s in range(_NSLOT):
        wait_store(s)


def kernel(trimap, alpha):
    t = trimap.reshape(_ROWS, _COLS)
    a = alpha.reshape(_ROWS, _COLS)
    out = _sc_fuse(t, a)
    return out.reshape(trimap.shape)
